# Initial kernel scaffold; baseline (speedup 1.0000x reference)
import jax
import jax.numpy as jnp
from jax.experimental import pallas as pl


def kernel(nodes, edge_index, W_in, b_in, W_neigh, b_neigh, ln_g, ln_b, W_out, b_out):
    raise NotImplementedError("write your pallas kernel here")


# speedup vs baseline: 3.7114x; 3.7114x over previous
import functools

import jax
import jax.numpy as jnp
from jax import lax
from jax.experimental import pallas as pl
from jax.experimental.pallas import tpu as pltpu
from jax.experimental.pallas import tpu_sc as plsc

N = 10000
E = 320000
D_IN = 128
H = 128
D_OUT = 64
LAYERS = 3

NW = 32
CH = 128
CHUNKS = 79
EPW = CHUNKS * CH
E_PAD = NW * EPW
N_ACC = 10112
RPT = N_ACC // 16
DW = 16

R = 1000

_mesh = plsc.VectorSubcoreMesh(core_axis_name="c", subcore_axis_name="s")


def _sc_agg_body(h, src3, dst3, zh, p_out, src_l, dst_l, msg, acc, sem):
    cid = lax.axis_index("c")
    sid = lax.axis_index("s")
    wid = cid * 16 + sid
    r0 = sid * RPT
    pltpu.sync_copy(zh.at[pl.ds(r0, RPT)], acc.at[pl.ds(r0, RPT)])
    pltpu.sync_copy(src3.at[wid], src_l)
    pltpu.sync_copy(dst3.at[wid], dst_l)
    plsc.subcore_barrier()

    def body(j, carry):
        pltpu.async_copy(h.at[src_l.at[j]], msg, sem).wait()
        pltpu.sync_copy(msg, acc.at[dst_l.at[j]], add=True)
        return carry

    lax.fori_loop(0, CHUNKS, body, 0)
    plsc.subcore_barrier()
    pltpu.sync_copy(acc.at[pl.ds(r0, RPT)], p_out.at[cid, pl.ds(r0, RPT)])


_sc_agg = functools.partial(
    pl.kernel,
    mesh=_mesh,
    out_type=jax.ShapeDtypeStruct((2, N_ACC, H), jnp.float32),
    scratch_types=[
        pltpu.VMEM((CHUNKS, CH), jnp.int32),
        pltpu.VMEM((CHUNKS, CH), jnp.int32),
        pltpu.VMEM((CH, H), jnp.float32),
        pltpu.VMEM_SHARED((N_ACC, H), jnp.float32),
        pltpu.SemaphoreType.DMA,
    ],
)(_sc_agg_body)


def _tc_in_body(x_ref, w_ref, b_ref, o_ref):
    o_ref[...] = jnp.dot(x_ref[...], w_ref[...],
                         preferred_element_type=jnp.float32,
                 precision=lax.Precision.HIGHEST) + b_ref[...]


def _tc_layer_body(h_ref, p_ref0, p_ref1, d_ref0, d_ref1, w_ref, b_ref,
                   g_ref, be_ref, o_ref):
    deg = d_ref0[0, :, 0:1] + d_ref1[0, :, 0:1]
    t = (h_ref[...] + p_ref0[0] + p_ref1[0]) / (deg + 1.0)
    z = jnp.dot(t, w_ref[...], preferred_element_type=jnp.float32,
                 precision=lax.Precision.HIGHEST) + b_ref[...]
    z = jnp.maximum(z, 0.0)
    mu = jnp.mean(z, axis=-1, keepdims=True)
    zc = z - mu
    var = jnp.mean(zc * zc, axis=-1, keepdims=True)
    o_ref[...] = zc * lax.rsqrt(var + 1e-5) * g_ref[...] + be_ref[...]


def _tc_last_body(h_ref, p_ref0, p_ref1, d_ref0, d_ref1, w_ref, b_ref,
                  g_ref, be_ref, wo_ref, bo_ref, o_ref):
    deg = d_ref0[0, :, 0:1] + d_ref1[0, :, 0:1]
    t = (h_ref[...] + p_ref0[0] + p_ref1[0]) / (deg + 1.0)
    z = jnp.dot(t, w_ref[...], preferred_element_type=jnp.float32,
                 precision=lax.Precision.HIGHEST) + b_ref[...]
    z = jnp.maximum(z, 0.0)
    mu = jnp.mean(z, axis=-1, keepdims=True)
    zc = z - mu
    var = jnp.mean(zc * zc, axis=-1, keepdims=True)
    hn = zc * lax.rsqrt(var + 1e-5) * g_ref[...] + be_ref[...]
    o_ref[...] = jnp.dot(hn, wo_ref[...],
                         preferred_element_type=jnp.float32,
                 precision=lax.Precision.HIGHEST) + bo_ref[...]


def _row_spec(w):
    return pl.BlockSpec((R, w), lambda i: (i, 0))


def _full_spec(shape):
    nd = len(shape)
    return pl.BlockSpec(shape, lambda i: (0,) * nd)


def _part_spec(core, w):
    return pl.BlockSpec((1, R, w), lambda i, c=core: (c, i, 0))


def _tc_in(x, w, b):
    return pl.pallas_call(
        _tc_in_body,
        grid=(N // R,),
        in_specs=[_row_spec(D_IN), _full_spec((D_IN, H)), _full_spec((1, H))],
        out_specs=_row_spec(H),
        out_shape=jax.ShapeDtypeStruct((N, H), jnp.float32),
    )(x, w, b)


def _tc_layer(h, p, deg, w, b, g, be):
    return pl.pallas_call(
        _tc_layer_body,
        grid=(N // R,),
        in_specs=[_row_spec(H), _part_spec(0, H), _part_spec(1, H),
                  _part_spec(0, H), _part_spec(1, H),
                  _full_spec((H, H)), _full_spec((1, H)),
                  _full_spec((1, H)), _full_spec((1, H))],
        out_specs=_row_spec(H),
        out_shape=jax.ShapeDtypeStruct((N, H), jnp.float32),
    )(h, p, p, deg, deg, w, b, g, be)


def _tc_last(h, p, deg, w, b, g, be, wo, bo):
    return pl.pallas_call(
        _tc_last_body,
        grid=(N // R,),
        in_specs=[_row_spec(H), _part_spec(0, H), _part_spec(1, H),
                  _part_spec(0, H), _part_spec(1, H),
                  _full_spec((H, H)), _full_spec((1, H)),
                  _full_spec((1, H)), _full_spec((1, H)),
                  _full_spec((H, D_OUT)), _full_spec((1, D_OUT))],
        out_specs=_row_spec(D_OUT),
        out_shape=jax.ShapeDtypeStruct((N, D_OUT), jnp.float32),
    )(h, p, p, deg, deg, w, b, g, be, wo, bo)


def kernel(nodes, edge_index, W_in, b_in, W_neigh, b_neigh, ln_g, ln_b,
           W_out, b_out):
    src = edge_index[0]
    dst = edge_index[1]
    pad = E_PAD - E
    src3 = jnp.concatenate([src, jnp.zeros((pad,), jnp.int32)]).reshape(NW, CHUNKS, CH)
    dst3 = jnp.concatenate([dst, jnp.full((pad,), N, jnp.int32)]).reshape(NW, CHUNKS, CH)
    zh = jnp.zeros((N_ACC, H), jnp.float32)
    ones_nh = jnp.ones((N, H), jnp.float32)

    h = _tc_in(nodes, W_in, b_in.reshape(1, H))
    deg = _sc_agg(ones_nh, src3, dst3, zh)
    zh1 = zh + jnp.minimum(deg[0, 0:1, 0:1], 0.0)
    p = _sc_agg(h, src3, dst3, zh1)
    h = _tc_layer(h, p, deg, W_neigh[0], b_neigh[0].reshape(1, H),
                  ln_g[0].reshape(1, H), ln_b[0].reshape(1, H))
    p = _sc_agg(h, src3, dst3, zh)
    h = _tc_layer(h, p, deg, W_neigh[1], b_neigh[1].reshape(1, H),
                  ln_g[1].reshape(1, H), ln_b[1].reshape(1, H))
    p = _sc_agg(h, src3, dst3, zh)
    return _tc_last(h, p, deg, W_neigh[2], b_neigh[2].reshape(1, H),
                    ln_g[2].reshape(1, H), ln_b[2].reshape(1, H),
                    W_out, b_out.reshape(1, D_OUT))


# speedup vs baseline: 4.6767x; 1.2601x over previous
import functools

import jax
import jax.numpy as jnp
from jax import lax
from jax.experimental import pallas as pl
from jax.experimental.pallas import tpu as pltpu
from jax.experimental.pallas import tpu_sc as plsc

N = 10000
E = 320000
D_IN = 128
H = 128
D_OUT = 64

NW = 32
CH = 128
CHUNKS = 79
EPW = CHUNKS * CH
E_PAD = NW * EPW
N_ACC = 10112
RPT = N_ACC // 16

R = 1000

_mesh = plsc.VectorSubcoreMesh(core_axis_name="c", subcore_axis_name="s")


def _idx_load(flat, base, j, ring, sem):
    off = pl.multiple_of(base + j * CH, CH)
    return pltpu.async_copy(flat.at[pl.ds(off, CH)], ring.at[j % 4], sem)


def _idx_wait(flat, base, j, ring, sem):
    off = pl.multiple_of(base + j * CH, CH)
    pltpu.make_async_copy(flat.at[pl.ds(off, CH)], ring.at[j % 4], sem).wait()


def _sc_agg_body(h, srcf, dstf, zh, p_out, sidx, didx, msg, acc, semi, semg):
    cid = lax.axis_index("c")
    sid = lax.axis_index("s")
    wid = cid * 16 + sid
    base = wid * EPW
    r0 = sid * RPT
    pltpu.sync_copy(zh.at[pl.ds(r0, RPT)], acc.at[pl.ds(r0, RPT)])
    _idx_load(srcf, base, 0, sidx, semi)
    _idx_load(dstf, base, 0, didx, semi)
    _idx_load(srcf, base, 1, sidx, semi)
    _idx_load(dstf, base, 1, didx, semi)
    plsc.subcore_barrier()
    _idx_wait(srcf, base, 0, sidx, semi)
    _idx_wait(dstf, base, 0, didx, semi)
    pltpu.async_copy(h.at[sidx.at[0]], msg.at[0], semg)

    def body(j, carry):
        pltpu.make_async_copy(h.at[sidx.at[j % 4]], msg.at[j % 2], semg).wait()
        _idx_wait(srcf, base, j + 1, sidx, semi)
        _idx_wait(dstf, base, j + 1, didx, semi)
        pltpu.async_copy(h.at[sidx.at[(j + 1) % 4]], msg.at[(j + 1) % 2], semg)

        @pl.when(j + 2 < CHUNKS)
        def _():
            _idx_load(srcf, base, j + 2, sidx, semi)
            _idx_load(dstf, base, j + 2, didx, semi)

        pltpu.sync_copy(msg.at[j % 2], acc.at[didx.at[j % 4]], add=True)
        return carry

    lax.fori_loop(0, CHUNKS - 1, body, 0)
    jl = CHUNKS - 1
    pltpu.make_async_copy(h.at[sidx.at[jl % 4]], msg.at[jl % 2], semg).wait()
    pltpu.sync_copy(msg.at[jl % 2], acc.at[didx.at[jl % 4]], add=True)
    plsc.subcore_barrier()
    pltpu.sync_copy(acc.at[pl.ds(r0, RPT)], p_out.at[cid, pl.ds(r0, RPT)])


def _sc_deg_body(dstf, zh, ones, deg_out, didx, onesv, acc, semi):
    cid = lax.axis_index("c")
    sid = lax.axis_index("s")
    wid = cid * 16 + sid
    base = wid * EPW
    r0 = sid * RPT
    pltpu.sync_copy(zh.at[pl.ds(r0, RPT)], acc.at[pl.ds(r0, RPT)])
    pltpu.sync_copy(ones, onesv)
    _idx_load(dstf, base, 0, didx, semi)
    _idx_load(dstf, base, 1, didx, semi)
    plsc.subcore_barrier()

    def body(j, carry):
        _idx_wait(dstf, base, j, didx, semi)

        @pl.when(j + 2 < CHUNKS)
        def _():
            _idx_load(dstf, base, j + 2, didx, semi)

        pltpu.sync_copy(onesv, acc.at[didx.at[j % 4]], add=True)
        return carry

    lax.fori_loop(0, CHUNKS, body, 0)
    plsc.subcore_barrier()
    pltpu.sync_copy(acc.at[pl.ds(r0, RPT)], deg_out.at[cid, pl.ds(r0, RPT)])


_sc_agg = functools.partial(
    pl.kernel,
    mesh=_mesh,
    out_type=jax.ShapeDtypeStruct((2, N_ACC, H), jnp.float32),
    scratch_types=[
        pltpu.VMEM((4, CH), jnp.int32),
        pltpu.VMEM((4, CH), jnp.int32),
        pltpu.VMEM((2, CH, H), jnp.float32),
        pltpu.VMEM_SHARED((N_ACC, H), jnp.float32),
        pltpu.SemaphoreType.DMA,
        pltpu.SemaphoreType.DMA,
    ],
)(_sc_agg_body)

_sc_deg = functools.partial(
    pl.kernel,
    mesh=_mesh,
    out_type=jax.ShapeDtypeStruct((2, N_ACC, H), jnp.float32),
    scratch_types=[
        pltpu.VMEM((4, CH), jnp.int32),
        pltpu.VMEM((CH, H), jnp.float32),
        pltpu.VMEM_SHARED((N_ACC, H), jnp.float32),
        pltpu.SemaphoreType.DMA,
    ],
)(_sc_deg_body)


def _tc_in_body(x_ref, w_ref, b_ref, o_ref):
    o_ref[...] = jnp.dot(x_ref[...], w_ref[...],
                         preferred_element_type=jnp.float32,
                         precision=lax.Precision.HIGHEST) + b_ref[...]


def _tc_layer_body(h_ref, p_ref0, p_ref1, d_ref0, d_ref1, w_ref, b_ref,
                   g_ref, be_ref, o_ref):
    deg = d_ref0[0, :, 0:1] + d_ref1[0, :, 0:1]
    t = (h_ref[...] + p_ref0[0] + p_ref1[0]) / (deg + 1.0)
    z = jnp.dot(t, w_ref[...], preferred_element_type=jnp.float32,
                precision=lax.Precision.HIGHEST) + b_ref[...]
    z = jnp.maximum(z, 0.0)
    mu = jnp.mean(z, axis=-1, keepdims=True)
    zc = z - mu
    var = jnp.mean(zc * zc, axis=-1, keepdims=True)
    o_ref[...] = zc * lax.rsqrt(var + 1e-5) * g_ref[...] + be_ref[...]


def _tc_last_body(h_ref, p_ref0, p_ref1, d_ref0, d_ref1, w_ref, b_ref,
                  g_ref, be_ref, wo_ref, bo_ref, o_ref):
    deg = d_ref0[0, :, 0:1] + d_ref1[0, :, 0:1]
    t = (h_ref[...] + p_ref0[0] + p_ref1[0]) / (deg + 1.0)
    z = jnp.dot(t, w_ref[...], preferred_element_type=jnp.float32,
                precision=lax.Precision.HIGHEST) + b_ref[...]
    z = jnp.maximum(z, 0.0)
    mu = jnp.mean(z, axis=-1, keepdims=True)
    zc = z - mu
    var = jnp.mean(zc * zc, axis=-1, keepdims=True)
    hn = zc * lax.rsqrt(var + 1e-5) * g_ref[...] + be_ref[...]
    o_ref[...] = jnp.dot(hn, wo_ref[...],
                         preferred_element_type=jnp.float32,
                         precision=lax.Precision.HIGHEST) + bo_ref[...]


def _row_spec(w):
    return pl.BlockSpec((R, w), lambda i: (i, 0))


def _full_spec(shape):
    nd = len(shape)
    return pl.BlockSpec(shape, lambda i: (0,) * nd)


def _part_spec(core, w):
    return pl.BlockSpec((1, R, w), lambda i, c=core: (c, i, 0))


def _tc_in(x, w, b):
    return pl.pallas_call(
        _tc_in_body,
        grid=(N // R,),
        in_specs=[_row_spec(D_IN), _full_spec((D_IN, H)), _full_spec((1, H))],
        out_specs=_row_spec(H),
        out_shape=jax.ShapeDtypeStruct((N, H), jnp.float32),
    )(x, w, b)


def _tc_layer(h, p, deg, w, b, g, be):
    return pl.pallas_call(
        _tc_layer_body,
        grid=(N // R,),
        in_specs=[_row_spec(H), _part_spec(0, H), _part_spec(1, H),
                  _part_spec(0, H), _part_spec(1, H),
                  _full_spec((H, H)), _full_spec((1, H)),
                  _full_spec((1, H)), _full_spec((1, H))],
        out_specs=_row_spec(H),
        out_shape=jax.ShapeDtypeStruct((N, H), jnp.float32),
    )(h, p, p, deg, deg, w, b, g, be)


def _tc_last(h, p, deg, w, b, g, be, wo, bo):
    return pl.pallas_call(
        _tc_last_body,
        grid=(N // R,),
        in_specs=[_row_spec(H), _part_spec(0, H), _part_spec(1, H),
                  _part_spec(0, H), _part_spec(1, H),
                  _full_spec((H, H)), _full_spec((1, H)),
                  _full_spec((1, H)), _full_spec((1, H)),
                  _full_spec((H, D_OUT)), _full_spec((1, D_OUT))],
        out_specs=_row_spec(D_OUT),
        out_shape=jax.ShapeDtypeStruct((N, D_OUT), jnp.float32),
    )(h, p, p, deg, deg, w, b, g, be, wo, bo)


def kernel(nodes, edge_index, W_in, b_in, W_neigh, b_neigh, ln_g, ln_b,
           W_out, b_out):
    pad = E_PAD - E
    srcf = jnp.concatenate([edge_index[0], jnp.zeros((pad,), jnp.int32)])
    dstf = jnp.concatenate([edge_index[1], jnp.full((pad,), N, jnp.int32)])
    zh = jnp.zeros((N_ACC, H), jnp.float32)
    ones = jnp.ones((CH, H), jnp.float32)

    h = _tc_in(nodes, W_in, b_in.reshape(1, H))
    deg = _sc_deg(dstf, zh, ones)
    zh1 = zh + jnp.minimum(deg[0, 0:1, 0:1], 0.0)
    p = _sc_agg(h, srcf, dstf, zh1)
    h = _tc_layer(h, p, deg, W_neigh[0], b_neigh[0].reshape(1, H),
                  ln_g[0].reshape(1, H), ln_b[0].reshape(1, H))
    p = _sc_agg(h, srcf, dstf, zh)
    h = _tc_layer(h, p, deg, W_neigh[1], b_neigh[1].reshape(1, H),
                  ln_g[1].reshape(1, H), ln_b[1].reshape(1, H))
    p = _sc_agg(h, srcf, dstf, zh)
    return _tc_last(h, p, deg, W_neigh[2], b_neigh[2].reshape(1, H),
                    ln_g[2].reshape(1, H), ln_b[2].reshape(1, H),
                    W_out, b_out.reshape(1, D_OUT))


# speedup vs baseline: 5.2560x; 1.1239x over previous
import functools

import jax
import jax.numpy as jnp
from jax import lax
from jax.experimental import pallas as pl
from jax.experimental.pallas import tpu as pltpu
from jax.experimental.pallas import tpu_sc as plsc

N = 10000
E = 320000
D_IN = 128
H = 128
D_OUT = 64

NW = 32
CH = 128
CHUNKS = 79
EPW = CHUNKS * CH
E_PAD = NW * EPW
CH_A = 110
CH_B = 2 * CHUNKS - CH_A
C0_TOT = 16 * CH_A * CH
N_ACC = 10112
RPT = N_ACC // 16

R = 1000

_mesh = plsc.VectorSubcoreMesh(core_axis_name="c", subcore_axis_name="s")


def _idx_load(flat, base, j, ring, sem):
    off = pl.multiple_of(base + j * CH, CH)
    return pltpu.async_copy(flat.at[pl.ds(off, CH)], ring.at[j % 4], sem)


def _idx_wait(flat, base, j, ring, sem):
    off = pl.multiple_of(base + j * CH, CH)
    pltpu.make_async_copy(flat.at[pl.ds(off, CH)], ring.at[j % 4], sem).wait()


def _sc_agg_body(h, srcf, dstf, zh, p_out, sidx, didx, msg, acc, semi, semg):
    cid = lax.axis_index("c")
    sid = lax.axis_index("s")
    nch = jnp.where(cid == 0, CH_A, CH_B)
    base = jnp.where(cid == 0, sid * (CH_A * CH), C0_TOT + sid * (CH_B * CH))
    r0 = sid * RPT
    pltpu.sync_copy(zh.at[pl.ds(r0, RPT)], acc.at[pl.ds(r0, RPT)])
    _idx_load(srcf, base, 0, sidx, semi)
    _idx_load(dstf, base, 0, didx, semi)
    _idx_load(srcf, base, 1, sidx, semi)
    _idx_load(dstf, base, 1, didx, semi)
    plsc.subcore_barrier()
    _idx_wait(srcf, base, 0, sidx, semi)
    _idx_wait(dstf, base, 0, didx, semi)
    pltpu.async_copy(h.at[sidx.at[0]], msg.at[0], semg)

    def body(j, carry):
        pltpu.make_async_copy(h.at[sidx.at[j % 4]], msg.at[j % 2], semg).wait()
        _idx_wait(srcf, base, j + 1, sidx, semi)
        _idx_wait(dstf, base, j + 1, didx, semi)
        pltpu.async_copy(h.at[sidx.at[(j + 1) % 4]], msg.at[(j + 1) % 2], semg)

        @pl.when(j + 2 < nch)
        def _():
            _idx_load(srcf, base, j + 2, sidx, semi)
            _idx_load(dstf, base, j + 2, didx, semi)

        pltpu.sync_copy(msg.at[j % 2], acc.at[didx.at[j % 4]], add=True)
        return carry

    lax.fori_loop(0, nch - 1, body, 0)
    jl = nch - 1
    pltpu.make_async_copy(h.at[sidx.at[jl % 4]], msg.at[jl % 2], semg).wait()
    pltpu.sync_copy(msg.at[jl % 2], acc.at[didx.at[jl % 4]], add=True)
    plsc.subcore_barrier()
    pltpu.sync_copy(acc.at[pl.ds(r0, RPT)], p_out.at[cid, pl.ds(r0, RPT)])


def _sc_deg_body(dstf, zh, ones, deg_out, didx, onesv, acc, semi):
    cid = lax.axis_index("c")
    sid = lax.axis_index("s")
    wid = cid * 16 + sid
    base = wid * EPW
    r0 = sid * RPT
    pltpu.sync_copy(zh.at[pl.ds(r0, RPT)], acc.at[pl.ds(r0, RPT)])
    pltpu.sync_copy(ones, onesv)
    _idx_load(dstf, base, 0, didx, semi)
    _idx_load(dstf, base, 1, didx, semi)
    plsc.subcore_barrier()

    def body(j, carry):
        _idx_wait(dstf, base, j, didx, semi)

        @pl.when(j + 2 < CHUNKS)
        def _():
            _idx_load(dstf, base, j + 2, didx, semi)

        pltpu.sync_copy(onesv, acc.at[didx.at[j % 4]], add=True)
        return carry

    lax.fori_loop(0, CHUNKS, body, 0)
    plsc.subcore_barrier()
    pltpu.sync_copy(acc.at[pl.ds(r0, RPT)], deg_out.at[cid, pl.ds(r0, RPT)])


_sc_agg = functools.partial(
    pl.kernel,
    mesh=_mesh,
    out_type=jax.ShapeDtypeStruct((2, N_ACC, H), jnp.float32),
    scratch_types=[
        pltpu.VMEM((4, CH), jnp.int32),
        pltpu.VMEM((4, CH), jnp.int32),
        pltpu.VMEM((2, CH, H), jnp.float32),
        pltpu.VMEM_SHARED((N_ACC, H), jnp.float32),
        pltpu.SemaphoreType.DMA,
        pltpu.SemaphoreType.DMA,
    ],
)(_sc_agg_body)

_sc_deg = functools.partial(
    pl.kernel,
    mesh=_mesh,
    out_type=jax.ShapeDtypeStruct((2, N_ACC, H), jnp.float32),
    scratch_types=[
        pltpu.VMEM((4, CH), jnp.int32),
        pltpu.VMEM((CH, H), jnp.float32),
        pltpu.VMEM_SHARED((N_ACC, H), jnp.float32),
        pltpu.SemaphoreType.DMA,
    ],
)(_sc_deg_body)


def _tc_in_body(x_ref, w_ref, b_ref, o_ref):
    o_ref[...] = jnp.dot(x_ref[...], w_ref[...],
                         preferred_element_type=jnp.float32,
                         precision=lax.Precision.HIGHEST) + b_ref[...]


def _tc_layer_body(h_ref, p_ref0, p_ref1, d_ref0, d_ref1, w_ref, b_ref,
                   g_ref, be_ref, o_ref):
    deg = d_ref0[0, :, 0:1] + d_ref1[0, :, 0:1]
    t = (h_ref[...] + p_ref0[0] + p_ref1[0]) / (deg + 1.0)
    z = jnp.dot(t, w_ref[...], preferred_element_type=jnp.float32,
                precision=lax.Precision.HIGHEST) + b_ref[...]
    z = jnp.maximum(z, 0.0)
    mu = jnp.mean(z, axis=-1, keepdims=True)
    zc = z - mu
    var = jnp.mean(zc * zc, axis=-1, keepdims=True)
    o_ref[...] = zc * lax.rsqrt(var + 1e-5) * g_ref[...] + be_ref[...]


def _tc_last_body(h_ref, p_ref0, p_ref1, d_ref0, d_ref1, w_ref, b_ref,
                  g_ref, be_ref, wo_ref, bo_ref, o_ref):
    deg = d_ref0[0, :, 0:1] + d_ref1[0, :, 0:1]
    t = (h_ref[...] + p_ref0[0] + p_ref1[0]) / (deg + 1.0)
    z = jnp.dot(t, w_ref[...], preferred_element_type=jnp.float32,
                precision=lax.Precision.HIGHEST) + b_ref[...]
    z = jnp.maximum(z, 0.0)
    mu = jnp.mean(z, axis=-1, keepdims=True)
    zc = z - mu
    var = jnp.mean(zc * zc, axis=-1, keepdims=True)
    hn = zc * lax.rsqrt(var + 1e-5) * g_ref[...] + be_ref[...]
    o_ref[...] = jnp.dot(hn, wo_ref[...],
                         preferred_element_type=jnp.float32,
                         precision=lax.Precision.HIGHEST) + bo_ref[...]


def _row_spec(w):
    return pl.BlockSpec((R, w), lambda i: (i, 0))


def _full_spec(shape):
    nd = len(shape)
    return pl.BlockSpec(shape, lambda i: (0,) * nd)


def _part_spec(core, w):
    return pl.BlockSpec((1, R, w), lambda i, c=core: (c, i, 0))


def _tc_in(x, w, b):
    return pl.pallas_call(
        _tc_in_body,
        grid=(N // R,),
        in_specs=[_row_spec(D_IN), _full_spec((D_IN, H)), _full_spec((1, H))],
        out_specs=_row_spec(H),
        out_shape=jax.ShapeDtypeStruct((N, H), jnp.float32),
    )(x, w, b)


def _tc_layer(h, p, deg, w, b, g, be):
    return pl.pallas_call(
        _tc_layer_body,
        grid=(N // R,),
        in_specs=[_row_spec(H), _part_spec(0, H), _part_spec(1, H),
                  _part_spec(0, H), _part_spec(1, H),
                  _full_spec((H, H)), _full_spec((1, H)),
                  _full_spec((1, H)), _full_spec((1, H))],
        out_specs=_row_spec(H),
        out_shape=jax.ShapeDtypeStruct((N, H), jnp.float32),
    )(h, p, p, deg, deg, w, b, g, be)


def _tc_last(h, p, deg, w, b, g, be, wo, bo):
    return pl.pallas_call(
        _tc_last_body,
        grid=(N // R,),
        in_specs=[_row_spec(H), _part_spec(0, H), _part_spec(1, H),
                  _part_spec(0, H), _part_spec(1, H),
                  _full_spec((H, H)), _full_spec((1, H)),
                  _full_spec((1, H)), _full_spec((1, H)),
                  _full_spec((H, D_OUT)), _full_spec((1, D_OUT))],
        out_specs=_row_spec(D_OUT),
        out_shape=jax.ShapeDtypeStruct((N, D_OUT), jnp.float32),
    )(h, p, p, deg, deg, w, b, g, be, wo, bo)


def kernel(nodes, edge_index, W_in, b_in, W_neigh, b_neigh, ln_g, ln_b,
           W_out, b_out):
    pad = E_PAD - E
    srcf = jnp.concatenate([edge_index[0], jnp.zeros((pad,), jnp.int32)])
    dstf = jnp.concatenate([edge_index[1], jnp.full((pad,), N, jnp.int32)])
    zh = jnp.zeros((N_ACC, H), jnp.float32)
    ones = jnp.ones((CH, H), jnp.float32)

    h = _tc_in(nodes, W_in, b_in.reshape(1, H))
    deg = _sc_deg(dstf, zh, ones)
    zh1 = zh + jnp.minimum(deg[0, 0:1, 0:1], 0.0)
    p = _sc_agg(h, srcf, dstf, zh1)
    h = _tc_layer(h, p, deg, W_neigh[0], b_neigh[0].reshape(1, H),
                  ln_g[0].reshape(1, H), ln_b[0].reshape(1, H))
    p = _sc_agg(h, srcf, dstf, zh)
    h = _tc_layer(h, p, deg, W_neigh[1], b_neigh[1].reshape(1, H),
                  ln_g[1].reshape(1, H), ln_b[1].reshape(1, H))
    p = _sc_agg(h, srcf, dstf, zh)
    return _tc_last(h, p, deg, W_neigh[2], b_neigh[2].reshape(1, H),
                    ln_g[2].reshape(1, H), ln_b[2].reshape(1, H),
                    W_out, b_out.reshape(1, D_OUT))


# speedup vs baseline: 5.7626x; 1.0964x over previous
import functools

import jax
import jax.numpy as jnp
from jax import lax
from jax.experimental import pallas as pl
from jax.experimental.pallas import tpu as pltpu
from jax.experimental.pallas import tpu_sc as plsc

N = 10000
E = 320000
D_IN = 128
H = 128
D_OUT = 64

NW = 32
CH = 128
CHUNKS = 79
EPW = CHUNKS * CH
E_PAD = NW * EPW
CH_A = 134
CH_B = 2 * CHUNKS - CH_A
C0_TOT = 16 * CH_A * CH
N_ACC = 10112
RPT = N_ACC // 16

R = 1000

_mesh = plsc.VectorSubcoreMesh(core_axis_name="c", subcore_axis_name="s")


def _idx_load(flat, base, j, ring, sem):
    off = pl.multiple_of(base + j * CH, CH)
    return pltpu.async_copy(flat.at[pl.ds(off, CH)], ring.at[j % 4], sem)


def _idx_wait(flat, base, j, ring, sem):
    off = pl.multiple_of(base + j * CH, CH)
    pltpu.make_async_copy(flat.at[pl.ds(off, CH)], ring.at[j % 4], sem).wait()


def _sc_agg_body(h, srcf, dstf, zh, p_out, sidx, didx, msg, acc, semi, semg):
    cid = lax.axis_index("c")
    sid = lax.axis_index("s")
    nch = jnp.where(cid == 0, CH_A, CH_B)
    base = jnp.where(cid == 0, sid * (CH_A * CH), C0_TOT + sid * (CH_B * CH))
    r0 = sid * RPT
    pltpu.sync_copy(zh.at[pl.ds(r0, RPT)], acc.at[pl.ds(r0, RPT)])
    _idx_load(srcf, base, 0, sidx, semi)
    _idx_load(dstf, base, 0, didx, semi)
    _idx_load(srcf, base, 1, sidx, semi)
    _idx_load(dstf, base, 1, didx, semi)
    plsc.subcore_barrier()
    _idx_wait(srcf, base, 0, sidx, semi)
    _idx_wait(dstf, base, 0, didx, semi)
    pltpu.async_copy(h.at[sidx.at[0]], msg.at[0], semg)

    def body(j, carry):
        pltpu.make_async_copy(h.at[sidx.at[j % 4]], msg.at[j % 2], semg).wait()
        _idx_wait(srcf, base, j + 1, sidx, semi)
        _idx_wait(dstf, base, j + 1, didx, semi)
        pltpu.async_copy(h.at[sidx.at[(j + 1) % 4]], msg.at[(j + 1) % 2], semg)

        @pl.when(j + 2 < nch)
        def _():
            _idx_load(srcf, base, j + 2, sidx, semi)
            _idx_load(dstf, base, j + 2, didx, semi)

        pltpu.sync_copy(msg.at[j % 2], acc.at[didx.at[j % 4]], add=True)
        return carry

    lax.fori_loop(0, nch - 1, body, 0)
    jl = nch - 1
    pltpu.make_async_copy(h.at[sidx.at[jl % 4]], msg.at[jl % 2], semg).wait()
    pltpu.sync_copy(msg.at[jl % 2], acc.at[didx.at[jl % 4]], add=True)
    plsc.subcore_barrier()
    pltpu.sync_copy(acc.at[pl.ds(r0, RPT)], p_out.at[cid, pl.ds(r0, RPT)])


def _sc_deg_body(dstf, zh, ones, deg_out, didx, onesv, acc, semi):
    cid = lax.axis_index("c")
    sid = lax.axis_index("s")
    wid = cid * 16 + sid
    base = wid * EPW
    r0 = sid * RPT
    pltpu.sync_copy(zh.at[pl.ds(r0, RPT)], acc.at[pl.ds(r0, RPT)])
    pltpu.sync_copy(ones, onesv)
    _idx_load(dstf, base, 0, didx, semi)
    _idx_load(dstf, base, 1, didx, semi)
    plsc.subcore_barrier()

    def body(j, carry):
        _idx_wait(dstf, base, j, didx, semi)

        @pl.when(j + 2 < CHUNKS)
        def _():
            _idx_load(dstf, base, j + 2, didx, semi)

        pltpu.sync_copy(onesv, acc.at[didx.at[j % 4]], add=True)
        return carry

    lax.fori_loop(0, CHUNKS, body, 0)
    plsc.subcore_barrier()
    pltpu.sync_copy(acc.at[pl.ds(r0, RPT)], deg_out.at[cid, pl.ds(r0, RPT)])


_sc_agg = functools.partial(
    pl.kernel,
    mesh=_mesh,
    out_type=jax.ShapeDtypeStruct((2, N_ACC, H), jnp.float32),
    scratch_types=[
        pltpu.VMEM((4, CH), jnp.int32),
        pltpu.VMEM((4, CH), jnp.int32),
        pltpu.VMEM((2, CH, H), jnp.float32),
        pltpu.VMEM_SHARED((N_ACC, H), jnp.float32),
        pltpu.SemaphoreType.DMA,
        pltpu.SemaphoreType.DMA,
    ],
)(_sc_agg_body)

_sc_deg = functools.partial(
    pl.kernel,
    mesh=_mesh,
    out_type=jax.ShapeDtypeStruct((2, N_ACC, H), jnp.float32),
    scratch_types=[
        pltpu.VMEM((4, CH), jnp.int32),
        pltpu.VMEM((CH, H), jnp.float32),
        pltpu.VMEM_SHARED((N_ACC, H), jnp.float32),
        pltpu.SemaphoreType.DMA,
    ],
)(_sc_deg_body)


def _tc_in_body(x_ref, w_ref, b_ref, o_ref):
    o_ref[...] = jnp.dot(x_ref[...], w_ref[...],
                         preferred_element_type=jnp.float32,
                         precision=lax.Precision.HIGHEST) + b_ref[...]


def _tc_layer_body(h_ref, p_ref0, p_ref1, d_ref0, d_ref1, w_ref, b_ref,
                   g_ref, be_ref, o_ref):
    deg = d_ref0[0, :, 0:1] + d_ref1[0, :, 0:1]
    t = (h_ref[...] + p_ref0[0] + p_ref1[0]) / (deg + 1.0)
    z = jnp.dot(t, w_ref[...], preferred_element_type=jnp.float32,
                precision=lax.Precision.HIGHEST) + b_ref[...]
    z = jnp.maximum(z, 0.0)
    mu = jnp.mean(z, axis=-1, keepdims=True)
    zc = z - mu
    var = jnp.mean(zc * zc, axis=-1, keepdims=True)
    o_ref[...] = zc * lax.rsqrt(var + 1e-5) * g_ref[...] + be_ref[...]


def _tc_last_body(h_ref, p_ref0, p_ref1, d_ref0, d_ref1, w_ref, b_ref,
                  g_ref, be_ref, wo_ref, bo_ref, o_ref):
    deg = d_ref0[0, :, 0:1] + d_ref1[0, :, 0:1]
    t = (h_ref[...] + p_ref0[0] + p_ref1[0]) / (deg + 1.0)
    z = jnp.dot(t, w_ref[...], preferred_element_type=jnp.float32,
                precision=lax.Precision.HIGHEST) + b_ref[...]
    z = jnp.maximum(z, 0.0)
    mu = jnp.mean(z, axis=-1, keepdims=True)
    zc = z - mu
    var = jnp.mean(zc * zc, axis=-1, keepdims=True)
    hn = zc * lax.rsqrt(var + 1e-5) * g_ref[...] + be_ref[...]
    o_ref[...] = jnp.dot(hn, wo_ref[...],
                         preferred_element_type=jnp.float32,
                         precision=lax.Precision.HIGHEST) + bo_ref[...]


def _row_spec(w):
    return pl.BlockSpec((R, w), lambda i: (i, 0))


def _full_spec(shape):
    nd = len(shape)
    return pl.BlockSpec(shape, lambda i: (0,) * nd)


def _part_spec(core, w):
    return pl.BlockSpec((1, R, w), lambda i, c=core: (c, i, 0))


def _tc_in(x, w, b):
    return pl.pallas_call(
        _tc_in_body,
        grid=(N // R,),
        in_specs=[_row_spec(D_IN), _full_spec((D_IN, H)), _full_spec((1, H))],
        out_specs=_row_spec(H),
        out_shape=jax.ShapeDtypeStruct((N, H), jnp.float32),
    )(x, w, b)


def _tc_layer(h, p, deg, w, b, g, be):
    return pl.pallas_call(
        _tc_layer_body,
        grid=(N // R,),
        in_specs=[_row_spec(H), _part_spec(0, H), _part_spec(1, H),
                  _part_spec(0, H), _part_spec(1, H),
                  _full_spec((H, H)), _full_spec((1, H)),
                  _full_spec((1, H)), _full_spec((1, H))],
        out_specs=_row_spec(H),
        out_shape=jax.ShapeDtypeStruct((N, H), jnp.float32),
    )(h, p, p, deg, deg, w, b, g, be)


def _tc_last(h, p, deg, w, b, g, be, wo, bo):
    return pl.pallas_call(
        _tc_last_body,
        grid=(N // R,),
        in_specs=[_row_spec(H), _part_spec(0, H), _part_spec(1, H),
                  _part_spec(0, H), _part_spec(1, H),
                  _full_spec((H, H)), _full_spec((1, H)),
                  _full_spec((1, H)), _full_spec((1, H)),
                  _full_spec((H, D_OUT)), _full_spec((1, D_OUT))],
        out_specs=_row_spec(D_OUT),
        out_shape=jax.ShapeDtypeStruct((N, D_OUT), jnp.float32),
    )(h, p, p, deg, deg, w, b, g, be, wo, bo)


def kernel(nodes, edge_index, W_in, b_in, W_neigh, b_neigh, ln_g, ln_b,
           W_out, b_out):
    pad = E_PAD - E
    srcf = jnp.concatenate([edge_index[0], jnp.zeros((pad,), jnp.int32)])
    dstf = jnp.concatenate([edge_index[1], jnp.full((pad,), N, jnp.int32)])
    zh = jnp.zeros((N_ACC, H), jnp.float32)
    ones = jnp.ones((CH, H), jnp.float32)

    h = _tc_in(nodes, W_in, b_in.reshape(1, H))
    deg = _sc_deg(dstf, zh, ones)
    zh1 = zh + jnp.minimum(deg[0, 0:1, 0:1], 0.0)
    p = _sc_agg(h, srcf, dstf, zh1)
    h = _tc_layer(h, p, deg, W_neigh[0], b_neigh[0].reshape(1, H),
                  ln_g[0].reshape(1, H), ln_b[0].reshape(1, H))
    p = _sc_agg(h, srcf, dstf, zh)
    h = _tc_layer(h, p, deg, W_neigh[1], b_neigh[1].reshape(1, H),
                  ln_g[1].reshape(1, H), ln_b[1].reshape(1, H))
    p = _sc_agg(h, srcf, dstf, zh)
    return _tc_last(h, p, deg, W_neigh[2], b_neigh[2].reshape(1, H),
                    ln_g[2].reshape(1, H), ln_b[2].reshape(1, H),
                    W_out, b_out.reshape(1, D_OUT))


# speedup vs baseline: 6.9014x; 1.1976x over previous
import functools

import jax
import jax.numpy as jnp
from jax import lax
from jax.experimental import pallas as pl
from jax.experimental.pallas import tpu as pltpu
from jax.experimental.pallas import tpu_sc as plsc

N = 10000
E = 320000
D_IN = 128
H = 128
D_OUT = 64

NW = 32
CH = 128
TOT = E // CH
CH_A = 134
C0_TOT = 16 * CH_A
CH_B = (TOT - C0_TOT) // 16
REM_B = (TOT - C0_TOT) % 16
CH_U = TOT // NW
REM_U = TOT % NW
N_ACC = 10112
RPT = N_ACC // 16

R = 1000

_mesh = plsc.VectorSubcoreMesh(core_axis_name="c", subcore_axis_name="s")


def _idx_load(flat, base, j, ring, sem):
    off = pl.multiple_of(base + j * CH, CH)
    return pltpu.async_copy(flat.at[pl.ds(off, CH)], ring.at[j % 4], sem)


def _idx_wait(flat, base, j, ring, sem):
    off = pl.multiple_of(base + j * CH, CH)
    pltpu.make_async_copy(flat.at[pl.ds(off, CH)], ring.at[j % 4], sem).wait()


def _zero_init(z8, msg0, acc, r0):
    pltpu.sync_copy(z8, msg0.at[pl.ds(0, 8)])
    zv = jnp.zeros((16,), jnp.float32)

    def zrow(i, c):
        for k in range(8):
            msg0[i, pl.ds(k * 16, 16)] = zv
        return c

    lax.fori_loop(0, CH, zrow, 0)
    for k in range(4):
        pltpu.sync_copy(msg0, acc.at[pl.ds(r0 + 128 * k, 128)])
    pltpu.sync_copy(msg0.at[pl.ds(0, RPT - 512)],
                    acc.at[pl.ds(r0 + 512, RPT - 512)])


def _sc_agg_body(h, srcf, dstf, z8, p_out, sidx, didx, msg, acc, semi, semg):
    cid = lax.axis_index("c")
    sid = lax.axis_index("s")
    nch = jnp.where(cid == 0, CH_A, CH_B + (sid < REM_B))
    bch = jnp.where(cid == 0, sid * CH_A,
                    C0_TOT + sid * CH_B + jnp.minimum(sid, REM_B))
    base = bch * CH
    r0 = sid * RPT
    _zero_init(z8, msg.at[0], acc, r0)
    _idx_load(srcf, base, 0, sidx, semi)
    _idx_load(dstf, base, 0, didx, semi)
    _idx_load(srcf, base, 1, sidx, semi)
    _idx_load(dstf, base, 1, didx, semi)
    plsc.subcore_barrier()
    _idx_wait(srcf, base, 0, sidx, semi)
    _idx_wait(dstf, base, 0, didx, semi)
    pltpu.async_copy(h.at[sidx.at[0]], msg.at[0], semg)

    def body(j, carry):
        pltpu.make_async_copy(h.at[sidx.at[j % 4]], msg.at[j % 2], semg).wait()
        _idx_wait(srcf, base, j + 1, sidx, semi)
        _idx_wait(dstf, base, j + 1, didx, semi)
        pltpu.async_copy(h.at[sidx.at[(j + 1) % 4]], msg.at[(j + 1) % 2], semg)

        @pl.when(j + 2 < nch)
        def _():
            _idx_load(srcf, base, j + 2, sidx, semi)
            _idx_load(dstf, base, j + 2, didx, semi)

        pltpu.sync_copy(msg.at[j % 2], acc.at[didx.at[j % 4]], add=True)
        return carry

    lax.fori_loop(0, nch - 1, body, 0)
    jl = nch - 1
    pltpu.make_async_copy(h.at[sidx.at[jl % 4]], msg.at[jl % 2], semg).wait()
    pltpu.sync_copy(msg.at[jl % 2], acc.at[didx.at[jl % 4]], add=True)
    plsc.subcore_barrier()
    pltpu.sync_copy(acc.at[pl.ds(r0, RPT)], p_out.at[cid, pl.ds(r0, RPT)])


def _sc_deg_body(dstf, z8, ones, deg_out, didx, onesv, zbuf, acc, semi):
    cid = lax.axis_index("c")
    sid = lax.axis_index("s")
    wid = cid * 16 + sid
    nch = CH_U + (wid < REM_U)
    base = (wid * CH_U + jnp.minimum(wid, REM_U)) * CH
    r0 = sid * RPT
    _zero_init(z8, zbuf, acc, r0)
    pltpu.sync_copy(ones, onesv)
    _idx_load(dstf, base, 0, didx, semi)
    _idx_load(dstf, base, 1, didx, semi)
    plsc.subcore_barrier()

    def body(j, carry):
        _idx_wait(dstf, base, j, didx, semi)

        @pl.when(j + 2 < nch)
        def _():
            _idx_load(dstf, base, j + 2, didx, semi)

        pltpu.sync_copy(onesv, acc.at[didx.at[j % 4]], add=True)
        return carry

    lax.fori_loop(0, nch, body, 0)
    plsc.subcore_barrier()
    pltpu.sync_copy(acc.at[pl.ds(r0, RPT)], deg_out.at[cid, pl.ds(r0, RPT)])


_sc_agg = functools.partial(
    pl.kernel,
    mesh=_mesh,
    out_type=jax.ShapeDtypeStruct((2, N_ACC, H), jnp.float32),
    scratch_types=[
        pltpu.VMEM((4, CH), jnp.int32),
        pltpu.VMEM((4, CH), jnp.int32),
        pltpu.VMEM((2, CH, H), jnp.float32),
        pltpu.VMEM_SHARED((N_ACC, H), jnp.float32),
        pltpu.SemaphoreType.DMA,
        pltpu.SemaphoreType.DMA,
    ],
)(_sc_agg_body)

_sc_deg = functools.partial(
    pl.kernel,
    mesh=_mesh,
    out_type=jax.ShapeDtypeStruct((2, N_ACC, H), jnp.float32),
    scratch_types=[
        pltpu.VMEM((4, CH), jnp.int32),
        pltpu.VMEM((CH, H), jnp.float32),
        pltpu.VMEM((CH, H), jnp.float32),
        pltpu.VMEM_SHARED((N_ACC, H), jnp.float32),
        pltpu.SemaphoreType.DMA,
    ],
)(_sc_deg_body)


def _tc_in_body(x_ref, w_ref, b_ref, o_ref):
    o_ref[...] = jnp.dot(x_ref[...], w_ref[...],
                         preferred_element_type=jnp.float32,
                         precision=lax.Precision.HIGHEST) + b_ref[...]


def _tc_layer_body(h_ref, p_ref0, p_ref1, d_ref0, d_ref1, w_ref, b_ref,
                   g_ref, be_ref, o_ref):
    deg = d_ref0[0, :, 0:1] + d_ref1[0, :, 0:1]
    t = (h_ref[...] + p_ref0[0] + p_ref1[0]) / (deg + 1.0)
    z = jnp.dot(t, w_ref[...], preferred_element_type=jnp.float32,
                precision=lax.Precision.HIGHEST) + b_ref[...]
    z = jnp.maximum(z, 0.0)
    mu = jnp.mean(z, axis=-1, keepdims=True)
    zc = z - mu
    var = jnp.mean(zc * zc, axis=-1, keepdims=True)
    o_ref[...] = zc * lax.rsqrt(var + 1e-5) * g_ref[...] + be_ref[...]


def _tc_last_body(h_ref, p_ref0, p_ref1, d_ref0, d_ref1, w_ref, b_ref,
                  g_ref, be_ref, wo_ref, bo_ref, o_ref):
    deg = d_ref0[0, :, 0:1] + d_ref1[0, :, 0:1]
    t = (h_ref[...] + p_ref0[0] + p_ref1[0]) / (deg + 1.0)
    z = jnp.dot(t, w_ref[...], preferred_element_type=jnp.float32,
                precision=lax.Precision.HIGHEST) + b_ref[...]
    z = jnp.maximum(z, 0.0)
    mu = jnp.mean(z, axis=-1, keepdims=True)
    zc = z - mu
    var = jnp.mean(zc * zc, axis=-1, keepdims=True)
    hn = zc * lax.rsqrt(var + 1e-5) * g_ref[...] + be_ref[...]
    o_ref[...] = jnp.dot(hn, wo_ref[...],
                         preferred_element_type=jnp.float32,
                         precision=lax.Precision.HIGHEST) + bo_ref[...]


def _row_spec(w):
    return pl.BlockSpec((R, w), lambda i: (i, 0))


def _full_spec(shape):
    nd = len(shape)
    return pl.BlockSpec(shape, lambda i: (0,) * nd)


def _part_spec(core, w):
    return pl.BlockSpec((1, R, w), lambda i, c=core: (c, i, 0))


def _tc_in(x, w, b):
    return pl.pallas_call(
        _tc_in_body,
        grid=(N // R,),
        in_specs=[_row_spec(D_IN), _full_spec((D_IN, H)), _full_spec((1, H))],
        out_specs=_row_spec(H),
        out_shape=jax.ShapeDtypeStruct((N, H), jnp.float32),
    )(x, w, b)


def _tc_layer(h, p, deg, w, b, g, be):
    return pl.pallas_call(
        _tc_layer_body,
        grid=(N // R,),
        in_specs=[_row_spec(H), _part_spec(0, H), _part_spec(1, H),
                  _part_spec(0, H), _part_spec(1, H),
                  _full_spec((H, H)), _full_spec((1, H)),
                  _full_spec((1, H)), _full_spec((1, H))],
        out_specs=_row_spec(H),
        out_shape=jax.ShapeDtypeStruct((N, H), jnp.float32),
    )(h, p, p, deg, deg, w, b, g, be)


def _tc_last(h, p, deg, w, b, g, be, wo, bo):
    return pl.pallas_call(
        _tc_last_body,
        grid=(N // R,),
        in_specs=[_row_spec(H), _part_spec(0, H), _part_spec(1, H),
                  _part_spec(0, H), _part_spec(1, H),
                  _full_spec((H, H)), _full_spec((1, H)),
                  _full_spec((1, H)), _full_spec((1, H)),
                  _full_spec((H, D_OUT)), _full_spec((1, D_OUT))],
        out_specs=_row_spec(D_OUT),
        out_shape=jax.ShapeDtypeStruct((N, D_OUT), jnp.float32),
    )(h, p, p, deg, deg, w, b, g, be, wo, bo)


def kernel(nodes, edge_index, W_in, b_in, W_neigh, b_neigh, ln_g, ln_b,
           W_out, b_out):
    srcf = edge_index[0]
    dstf = edge_index[1]
    z8 = jnp.zeros((8, H), jnp.float32)
    ones = jnp.ones((CH, H), jnp.float32)

    h = _tc_in(nodes, W_in, b_in.reshape(1, H))
    deg = _sc_deg(dstf, z8, ones)
    z81 = z8 + jnp.minimum(deg[0, 0:1, 0:1], 0.0)
    p = _sc_agg(h, srcf, dstf, z81)
    h = _tc_layer(h, p, deg, W_neigh[0], b_neigh[0].reshape(1, H),
                  ln_g[0].reshape(1, H), ln_b[0].reshape(1, H))
    p = _sc_agg(h, srcf, dstf, z8)
    h = _tc_layer(h, p, deg, W_neigh[1], b_neigh[1].reshape(1, H),
                  ln_g[1].reshape(1, H), ln_b[1].reshape(1, H))
    p = _sc_agg(h, srcf, dstf, z8)
    return _tc_last(h, p, deg, W_neigh[2], b_neigh[2].reshape(1, H),
                    ln_g[2].reshape(1, H), ln_b[2].reshape(1, H),
                    W_out, b_out.reshape(1, D_OUT))


# speedup vs baseline: 8.9296x; 1.2939x over previous
import functools

import jax
import jax.numpy as jnp
from jax import lax
from jax.experimental import pallas as pl
from jax.experimental.pallas import tpu as pltpu
from jax.experimental.pallas import tpu_sc as plsc

N = 10000
E = 320000
D_IN = 128
H = 128
D_OUT = 64

NW = 32
CH = 128
TOT = E // CH
CH_A = 88
C0_TOT = 16 * CH_A
CH_B = (TOT - C0_TOT) // 16
REM_B = (TOT - C0_TOT) % 16
CH_U = TOT // NW
REM_U = TOT % NW
N_ACC = 10112
RPT = N_ACC // 16

R = 1000

_mesh = plsc.VectorSubcoreMesh(core_axis_name="c", subcore_axis_name="s")


def _idx_load(flat, base, j, ring, sem):
    off = pl.multiple_of(base + j * CH, CH)
    return pltpu.async_copy(flat.at[pl.ds(off, CH)], ring.at[j % 4], sem)


def _idx_wait(flat, base, j, ring, sem):
    off = pl.multiple_of(base + j * CH, CH)
    pltpu.make_async_copy(flat.at[pl.ds(off, CH)], ring.at[j % 4], sem).wait()


def _zero_init(z8, msg0, acc, r0):
    pltpu.sync_copy(z8, msg0.at[pl.ds(0, 8)])
    zv = jnp.zeros((16,), jnp.float32)

    def zrow(i, c):
        for k in range(8):
            msg0[i, pl.ds(k * 16, 16)] = zv
        return c

    lax.fori_loop(0, CH, zrow, 0)
    for k in range(4):
        pltpu.sync_copy(msg0, acc.at[pl.ds(r0 + 128 * k, 128)])
    pltpu.sync_copy(msg0.at[pl.ds(0, RPT - 512)],
                    acc.at[pl.ds(r0 + 512, RPT - 512)])


def _sc_agg_body(h, srcf, dstf, z8, p_out, sidx, didx, msg, acc, semi, semg):
    cid = lax.axis_index("c")
    sid = lax.axis_index("s")
    nch = jnp.where(cid == 0, CH_A, CH_B + (sid < REM_B))
    bch = jnp.where(cid == 0, sid * CH_A,
                    C0_TOT + sid * CH_B + jnp.minimum(sid, REM_B))
    base = bch * CH
    r0 = sid * RPT
    _zero_init(z8, msg.at[0], acc, r0)
    _idx_load(srcf, base, 0, sidx, semi)
    _idx_load(dstf, base, 0, didx, semi)
    _idx_load(srcf, base, 1, sidx, semi)
    _idx_load(dstf, base, 1, didx, semi)
    plsc.subcore_barrier()
    _idx_wait(srcf, base, 0, sidx, semi)
    _idx_wait(dstf, base, 0, didx, semi)
    pltpu.async_copy(h.at[sidx.at[0]], msg.at[0], semg)

    def body(j, carry):
        pltpu.make_async_copy(h.at[sidx.at[j % 4]], msg.at[j % 2], semg).wait()
        _idx_wait(srcf, base, j + 1, sidx, semi)
        _idx_wait(dstf, base, j + 1, didx, semi)
        pltpu.async_copy(h.at[sidx.at[(j + 1) % 4]], msg.at[(j + 1) % 2], semg)

        @pl.when(j + 2 < nch)
        def _():
            _idx_load(srcf, base, j + 2, sidx, semi)
            _idx_load(dstf, base, j + 2, didx, semi)

        pltpu.sync_copy(msg.at[j % 2], acc.at[didx.at[j % 4]], add=True)
        return carry

    lax.fori_loop(0, nch - 1, body, 0)
    jl = nch - 1
    pltpu.make_async_copy(h.at[sidx.at[jl % 4]], msg.at[jl % 2], semg).wait()
    pltpu.sync_copy(msg.at[jl % 2], acc.at[didx.at[jl % 4]], add=True)
    plsc.subcore_barrier()
    pltpu.sync_copy(acc.at[pl.ds(r0, RPT)], p_out.at[cid, pl.ds(r0, RPT)])


def _sc_deg_body(dstf, z8, ones, deg_out, didx, onesv, zbuf, acc, semi):
    cid = lax.axis_index("c")
    sid = lax.axis_index("s")
    wid = cid * 16 + sid
    nch = CH_U + (wid < REM_U)
    base = (wid * CH_U + jnp.minimum(wid, REM_U)) * CH
    r0 = sid * RPT
    _zero_init(z8, zbuf, acc, r0)
    pltpu.sync_copy(ones, onesv)
    _idx_load(dstf, base, 0, didx, semi)
    _idx_load(dstf, base, 1, didx, semi)
    plsc.subcore_barrier()

    def body(j, carry):
        _idx_wait(dstf, base, j, didx, semi)

        @pl.when(j + 2 < nch)
        def _():
            _idx_load(dstf, base, j + 2, didx, semi)

        pltpu.sync_copy(onesv, acc.at[didx.at[j % 4]], add=True)
        return carry

    lax.fori_loop(0, nch, body, 0)
    plsc.subcore_barrier()
    pltpu.sync_copy(acc.at[pl.ds(r0, RPT)], deg_out.at[cid, pl.ds(r0, RPT)])


_sc_agg = functools.partial(
    pl.kernel,
    mesh=_mesh,
    out_type=jax.ShapeDtypeStruct((2, N_ACC, H), jnp.float32),
    scratch_types=[
        pltpu.VMEM((4, CH), jnp.int32),
        pltpu.VMEM((4, CH), jnp.int32),
        pltpu.VMEM((2, CH, H), jnp.float32),
        pltpu.VMEM_SHARED((N_ACC, H), jnp.float32),
        pltpu.SemaphoreType.DMA,
        pltpu.SemaphoreType.DMA,
    ],
)(_sc_agg_body)

_sc_deg = functools.partial(
    pl.kernel,
    mesh=_mesh,
    out_type=jax.ShapeDtypeStruct((2, N_ACC, H), jnp.float32),
    scratch_types=[
        pltpu.VMEM((4, CH), jnp.int32),
        pltpu.VMEM((CH, H), jnp.float32),
        pltpu.VMEM((CH, H), jnp.float32),
        pltpu.VMEM_SHARED((N_ACC, H), jnp.float32),
        pltpu.SemaphoreType.DMA,
    ],
)(_sc_deg_body)


def _tc_in_body(x_ref, w_ref, b_ref, o_ref):
    o_ref[...] = jnp.dot(x_ref[...], w_ref[...],
                         preferred_element_type=jnp.float32,
                         precision=lax.Precision.HIGHEST) + b_ref[...]


def _tc_layer_body(h_ref, p_ref0, p_ref1, d_ref0, d_ref1, w_ref, b_ref,
                   g_ref, be_ref, o_ref):
    deg = d_ref0[0, :, 0:1] + d_ref1[0, :, 0:1]
    t = (h_ref[...] + p_ref0[0] + p_ref1[0]) / (deg + 1.0)
    z = jnp.dot(t, w_ref[...], preferred_element_type=jnp.float32,
                precision=lax.Precision.HIGHEST) + b_ref[...]
    z = jnp.maximum(z, 0.0)
    mu = jnp.mean(z, axis=-1, keepdims=True)
    zc = z - mu
    var = jnp.mean(zc * zc, axis=-1, keepdims=True)
    o_ref[...] = zc * lax.rsqrt(var + 1e-5) * g_ref[...] + be_ref[...]


def _tc_last_body(h_ref, p_ref0, p_ref1, d_ref0, d_ref1, w_ref, b_ref,
                  g_ref, be_ref, wo_ref, bo_ref, o_ref):
    deg = d_ref0[0, :, 0:1] + d_ref1[0, :, 0:1]
    t = (h_ref[...] + p_ref0[0] + p_ref1[0]) / (deg + 1.0)
    z = jnp.dot(t, w_ref[...], preferred_element_type=jnp.float32,
                precision=lax.Precision.HIGHEST) + b_ref[...]
    z = jnp.maximum(z, 0.0)
    mu = jnp.mean(z, axis=-1, keepdims=True)
    zc = z - mu
    var = jnp.mean(zc * zc, axis=-1, keepdims=True)
    hn = zc * lax.rsqrt(var + 1e-5) * g_ref[...] + be_ref[...]
    o_ref[...] = jnp.dot(hn, wo_ref[...],
                         preferred_element_type=jnp.float32,
                         precision=lax.Precision.HIGHEST) + bo_ref[...]


def _row_spec(w):
    return pl.BlockSpec((R, w), lambda i: (i, 0))


def _full_spec(shape):
    nd = len(shape)
    return pl.BlockSpec(shape, lambda i: (0,) * nd)


def _part_spec(core, w):
    return pl.BlockSpec((1, R, w), lambda i, c=core: (c, i, 0))


def _tc_in(x, w, b):
    return pl.pallas_call(
        _tc_in_body,
        grid=(N // R,),
        in_specs=[_row_spec(D_IN), _full_spec((D_IN, H)), _full_spec((1, H))],
        out_specs=_row_spec(H),
        out_shape=jax.ShapeDtypeStruct((N, H), jnp.float32),
    )(x, w, b)


def _tc_layer(h, p, deg, w, b, g, be):
    return pl.pallas_call(
        _tc_layer_body,
        grid=(N // R,),
        in_specs=[_row_spec(H), _part_spec(0, H), _part_spec(1, H),
                  _part_spec(0, H), _part_spec(1, H),
                  _full_spec((H, H)), _full_spec((1, H)),
                  _full_spec((1, H)), _full_spec((1, H))],
        out_specs=_row_spec(H),
        out_shape=jax.ShapeDtypeStruct((N, H), jnp.float32),
    )(h, p, p, deg, deg, w, b, g, be)


def _tc_last(h, p, deg, w, b, g, be, wo, bo):
    return pl.pallas_call(
        _tc_last_body,
        grid=(N // R,),
        in_specs=[_row_spec(H), _part_spec(0, H), _part_spec(1, H),
                  _part_spec(0, H), _part_spec(1, H),
                  _full_spec((H, H)), _full_spec((1, H)),
                  _full_spec((1, H)), _full_spec((1, H)),
                  _full_spec((H, D_OUT)), _full_spec((1, D_OUT))],
        out_specs=_row_spec(D_OUT),
        out_shape=jax.ShapeDtypeStruct((N, D_OUT), jnp.float32),
    )(h, p, p, deg, deg, w, b, g, be, wo, bo)


def kernel(nodes, edge_index, W_in, b_in, W_neigh, b_neigh, ln_g, ln_b,
           W_out, b_out):
    srcf = edge_index[0]
    dstf = edge_index[1]
    z8 = jnp.zeros((8, H), jnp.float32)
    ones = jnp.ones((CH, H), jnp.float32)

    h = _tc_in(nodes, W_in, b_in.reshape(1, H))
    deg = _sc_deg(dstf, z8, ones)
    z81 = z8 + jnp.minimum(deg[0, 0:1, 0:1], 0.0)
    p = _sc_agg(h, srcf, dstf, z81)
    h = _tc_layer(h, p, deg, W_neigh[0], b_neigh[0].reshape(1, H),
                  ln_g[0].reshape(1, H), ln_b[0].reshape(1, H))
    p = _sc_agg(h, srcf, dstf, z8)
    h = _tc_layer(h, p, deg, W_neigh[1], b_neigh[1].reshape(1, H),
                  ln_g[1].reshape(1, H), ln_b[1].reshape(1, H))
    p = _sc_agg(h, srcf, dstf, z8)
    return _tc_last(h, p, deg, W_neigh[2], b_neigh[2].reshape(1, H),
                    ln_g[2].reshape(1, H), ln_b[2].reshape(1, H),
                    W_out, b_out.reshape(1, D_OUT))
